# baseline (speedup 1.0000x reference)
import functools

import jax
import jax.numpy as jnp
from jax import lax
from jax.experimental import pallas as pl
from jax.experimental.pallas import tpu as pltpu
from jax.experimental.pallas import tpu_sc as plsc

EMBEDDING_DIM = 50
CONTEXT_SIZE = 5
NUM_CORES = 2
NUM_SUBCORES = 16
NUM_WORKERS = NUM_CORES * NUM_SUBCORES
CHUNK = 256
VEC = 16


@functools.partial(jax.jit, static_argnames=("n_chunks", "sizes"))
def _sc_gather_sum(emb_f, p_f, s_f, gi, pi, si, n_chunks, sizes):
    v_g, v_p, v_s = sizes
    dim = EMBEDDING_DIM
    tile = dim * CHUNK
    mesh = plsc.VectorSubcoreMesh(core_axis_name="c", subcore_axis_name="s")

    @functools.partial(
        pl.kernel,
        mesh=mesh,
        out_type=jax.ShapeDtypeStruct((NUM_WORKERS, n_chunks, tile),
                                      jnp.float32),
        compiler_params=pltpu.CompilerParams(use_tc_tiling_on_sc=False),
        scratch_types=[
            pltpu.VMEM((n_chunks, CHUNK), jnp.int32),
            pltpu.VMEM((n_chunks, CHUNK), jnp.int32),
            pltpu.VMEM((n_chunks, CHUNK), jnp.int32),
            pltpu.VMEM((tile,), jnp.int32),
            pltpu.VMEM((tile,), jnp.int32),
            pltpu.VMEM((tile,), jnp.int32),
            pltpu.VMEM((tile,), jnp.float32),
            pltpu.VMEM((tile,), jnp.float32),
            pltpu.VMEM((tile,), jnp.float32),
            pltpu.SemaphoreType.DMA,
            pltpu.SemaphoreType.DMA,
            pltpu.SemaphoreType.DMA,
            pltpu.SemaphoreType.DMA,
        ],
    )
    def k(emb_h, p_h, s_h, gi_h, pi_h, si_h, x_h,
          gi_v, pi_v, si_v, gix, pix, six, gbuf, pbuf, sbuf,
          sem0, sem1, sem2, sem3):
        wid = lax.axis_index("s") * NUM_CORES + lax.axis_index("c")
        pltpu.sync_copy(gi_h.at[wid], gi_v)
        pltpu.sync_copy(pi_h.at[wid], pi_v)
        pltpu.sync_copy(si_h.at[wid], si_v)

        def body(j, carry):
            def prep_body(d, _):
                for t in range(CHUNK // VEC):
                    src = pl.ds(t * VEC, VEC)
                    dst = pl.ds(d * CHUNK + t * VEC, VEC)
                    gix[dst] = gi_v[j, src] + d * v_g
                    pix[dst] = pi_v[j, src] + d * v_p
                    six[dst] = si_v[j, src] + d * v_s
                return _
            lax.fori_loop(0, dim, prep_body, 0)
            h0 = pltpu.async_copy(emb_h.at[gix], gbuf, sem0)
            h1 = pltpu.async_copy(p_h.at[pix], pbuf, sem1)
            h2 = pltpu.async_copy(s_h.at[six], sbuf, sem2)
            h0.wait()
            h1.wait()
            h2.wait()

            def sum_body(i, _):
                c = pl.ds(i * VEC, VEC)
                gbuf[c] = gbuf[c] + pbuf[c] + sbuf[c]
                return _
            lax.fori_loop(0, tile // VEC, sum_body, 0)
            o = pltpu.async_copy(gbuf, x_h.at[wid].at[j], sem3)
            o.wait()
            return carry

        lax.fori_loop(0, n_chunks, body, 0)

    return k(emb_f, p_f, s_f, gi, pi, si)


def _mlp_body(x_ref, w1_ref, b1_ref, w2_ref, b2_ref, o_ref):
    acc = None
    for c in range(CONTEXT_SIZE):
        part = jnp.dot(w1_ref[c], x_ref[c, 0],
                       preferred_element_type=jnp.float32)
        acc = part if acc is None else acc + part
    h = jnp.maximum(acc + b1_ref[...], 0.0)
    o = jnp.dot(w2_ref[...], h, preferred_element_type=jnp.float32)
    o = o + b2_ref[...]
    m = jnp.max(o, axis=0, keepdims=True)
    e = jnp.exp(o - m)
    lse = jnp.log(jnp.sum(e, axis=0, keepdims=True))
    o_ref[...] = (o - m) - lse


@jax.jit
def _tc_mlp(X5, W1t, b1, W2t, b2):
    ctx, n_tiles, dim, chunk = X5.shape
    hidden = W1t.shape[1]
    n_tags = W2t.shape[0]
    return pl.pallas_call(
        _mlp_body,
        grid=(n_tiles,),
        in_specs=[
            pl.BlockSpec((ctx, 1, dim, chunk), lambda i: (0, i, 0, 0)),
            pl.BlockSpec((ctx, hidden, dim), lambda i: (0, 0, 0)),
            pl.BlockSpec((hidden, 1), lambda i: (0, 0)),
            pl.BlockSpec((n_tags, hidden), lambda i: (0, 0)),
            pl.BlockSpec((n_tags, 1), lambda i: (0, 0)),
        ],
        out_specs=pl.BlockSpec((n_tags, chunk), lambda i: (0, i)),
        out_shape=jax.ShapeDtypeStruct((n_tags, n_tiles * chunk),
                                       jnp.float32),
    )(X5, W1t, b1, W2t, b2)


def kernel(inputs, p_inputs, s_inputs, emb, p_emb, s_emb, W1, b1, W2, b2):
    ctx, batch = inputs.shape
    dim = emb.shape[1]
    n_cols = ctx * batch
    per_w = n_cols // NUM_WORKERS
    n_chunks = per_w // CHUNK

    def prep(ix):
        return ix.reshape(NUM_WORKERS, n_chunks, CHUNK).astype(jnp.int32)

    def flat(tab):
        return tab.T.reshape(-1), tab.shape[0]

    emb_f, v_g = flat(emb)
    p_f, v_p = flat(p_emb)
    s_f, v_s = flat(s_emb)
    X = _sc_gather_sum(emb_f, p_f, s_f,
                       prep(inputs), prep(p_inputs), prep(s_inputs),
                       n_chunks, (v_g, v_p, v_s))
    X5 = X.reshape(ctx, batch // CHUNK, dim, CHUNK)
    W1t = W1.T.reshape(W1.shape[1], ctx, dim).transpose(1, 0, 2)
    oT = _tc_mlp(X5, W1t, b1.reshape(-1, 1), W2.T, b2.reshape(-1, 1))
    return oT.T


# speedup vs baseline: 4.4768x; 4.4768x over previous
import functools

import jax
import jax.numpy as jnp
from jax import lax
from jax.experimental import pallas as pl
from jax.experimental.pallas import tpu as pltpu
from jax.experimental.pallas import tpu_sc as plsc

EMBEDDING_DIM = 50
CONTEXT_SIZE = 5
NUM_CORES = 2
NUM_SUBCORES = 16
NUM_WORKERS = NUM_CORES * NUM_SUBCORES
CHUNK = 256
VEC = 16
FLAT_BLK = 65536


def _flatten_body(x_ref, o_ref):
    r = pl.program_id(1)
    o_ref[...] = x_ref[r]


@jax.jit
def _flatten(tabT):
    dim, v = tabT.shape
    nblk = (v + FLAT_BLK - 1) // FLAT_BLK
    vpad = nblk * FLAT_BLK
    return pl.pallas_call(
        _flatten_body,
        grid=(nblk, dim),
        in_specs=[pl.BlockSpec((dim, FLAT_BLK), lambda c, r: (0, c))],
        out_specs=pl.BlockSpec((FLAT_BLK,), lambda c, r: (r * nblk + c)),
        out_shape=jax.ShapeDtypeStruct((dim * vpad,), jnp.float32),
    )(tabT)


@functools.partial(jax.jit, static_argnames=("n_chunks", "sizes"))
def _sc_gather_sum(emb_f, p_f, s_f, gi, pi, si, n_chunks, sizes):
    v_g, v_p, v_s = sizes
    dim = EMBEDDING_DIM
    tile = dim * CHUNK
    mesh = plsc.VectorSubcoreMesh(core_axis_name="c", subcore_axis_name="s")

    @functools.partial(
        pl.kernel,
        mesh=mesh,
        out_type=jax.ShapeDtypeStruct((NUM_WORKERS, n_chunks, tile),
                                      jnp.float32),
        compiler_params=pltpu.CompilerParams(use_tc_tiling_on_sc=False),
        scratch_types=[
            pltpu.VMEM((n_chunks, CHUNK), jnp.int32),
            pltpu.VMEM((n_chunks, CHUNK), jnp.int32),
            pltpu.VMEM((n_chunks, CHUNK), jnp.int32),
            pltpu.VMEM((tile,), jnp.int32),
            pltpu.VMEM((tile,), jnp.int32),
            pltpu.VMEM((tile,), jnp.int32),
            pltpu.VMEM((tile,), jnp.float32),
            pltpu.VMEM((tile,), jnp.float32),
            pltpu.VMEM((tile,), jnp.float32),
            pltpu.SemaphoreType.DMA,
            pltpu.SemaphoreType.DMA,
            pltpu.SemaphoreType.DMA,
            pltpu.SemaphoreType.DMA,
        ],
    )
    def k(emb_h, p_h, s_h, gi_h, pi_h, si_h, x_h,
          gi_v, pi_v, si_v, gix, pix, six, gbuf, pbuf, sbuf,
          sem0, sem1, sem2, sem3):
        wid = lax.axis_index("s") * NUM_CORES + lax.axis_index("c")
        pltpu.sync_copy(gi_h.at[wid], gi_v)
        pltpu.sync_copy(pi_h.at[wid], pi_v)
        pltpu.sync_copy(si_h.at[wid], si_v)

        def body(j, carry):
            def prep_body(d, _):
                for t in range(CHUNK // VEC):
                    src = pl.ds(t * VEC, VEC)
                    dst = pl.ds(d * CHUNK + t * VEC, VEC)
                    gix[dst] = gi_v[j, src] + d * v_g
                    pix[dst] = pi_v[j, src] + d * v_p
                    six[dst] = si_v[j, src] + d * v_s
                return _
            lax.fori_loop(0, dim, prep_body, 0)
            h0 = pltpu.async_copy(emb_h.at[gix], gbuf, sem0)
            h1 = pltpu.async_copy(p_h.at[pix], pbuf, sem1)
            h2 = pltpu.async_copy(s_h.at[six], sbuf, sem2)
            h0.wait()
            h1.wait()
            h2.wait()

            def sum_body(i, _):
                c = pl.ds(i * VEC, VEC)
                gbuf[c] = gbuf[c] + pbuf[c] + sbuf[c]
                return _
            lax.fori_loop(0, tile // VEC, sum_body, 0)
            o = pltpu.async_copy(gbuf, x_h.at[wid].at[j], sem3)
            o.wait()
            return carry

        lax.fori_loop(0, n_chunks, body, 0)

    return k(emb_f, p_f, s_f, gi, pi, si)


def _mlp_body(x_ref, w1_ref, b1_ref, w2_ref, b2_ref, o_ref):
    acc = None
    for c in range(CONTEXT_SIZE):
        part = jnp.dot(w1_ref[c], x_ref[c, 0],
                       preferred_element_type=jnp.float32)
        acc = part if acc is None else acc + part
    h = jnp.maximum(acc + b1_ref[...], 0.0)
    o = jnp.dot(w2_ref[...], h, preferred_element_type=jnp.float32)
    o = o + b2_ref[...]
    m = jnp.max(o, axis=0, keepdims=True)
    e = jnp.exp(o - m)
    lse = jnp.log(jnp.sum(e, axis=0, keepdims=True))
    o_ref[...] = (o - m) - lse


@jax.jit
def _tc_mlp(X5, W1t, b1, W2t, b2):
    ctx, n_tiles, dim, chunk = X5.shape
    hidden = W1t.shape[1]
    n_tags = W2t.shape[0]
    return pl.pallas_call(
        _mlp_body,
        grid=(n_tiles,),
        in_specs=[
            pl.BlockSpec((ctx, 1, dim, chunk), lambda i: (0, i, 0, 0)),
            pl.BlockSpec((ctx, hidden, dim), lambda i: (0, 0, 0)),
            pl.BlockSpec((hidden, 1), lambda i: (0, 0)),
            pl.BlockSpec((n_tags, hidden), lambda i: (0, 0)),
            pl.BlockSpec((n_tags, 1), lambda i: (0, 0)),
        ],
        out_specs=pl.BlockSpec((n_tags, chunk), lambda i: (0, i)),
        out_shape=jax.ShapeDtypeStruct((n_tags, n_tiles * chunk),
                                       jnp.float32),
    )(X5, W1t, b1, W2t, b2)


def kernel(inputs, p_inputs, s_inputs, emb, p_emb, s_emb, W1, b1, W2, b2):
    ctx, batch = inputs.shape
    dim = emb.shape[1]
    n_cols = ctx * batch
    per_w = n_cols // NUM_WORKERS
    n_chunks = per_w // CHUNK

    def prep(ix):
        return ix.reshape(NUM_WORKERS, n_chunks, CHUNK).astype(jnp.int32)

    def flat(tab):
        v = tab.shape[0]
        vpad = (v + FLAT_BLK - 1) // FLAT_BLK * FLAT_BLK
        return _flatten(tab.T), vpad

    emb_f, v_g = flat(emb)
    p_f, v_p = flat(p_emb)
    s_f, v_s = flat(s_emb)
    X = _sc_gather_sum(emb_f, p_f, s_f,
                       prep(inputs), prep(p_inputs), prep(s_inputs),
                       n_chunks, (v_g, v_p, v_s))
    X5 = X.reshape(ctx, batch // CHUNK, dim, CHUNK)
    W1t = W1.T.reshape(W1.shape[1], ctx, dim).transpose(1, 0, 2)
    oT = _tc_mlp(X5, W1t, b1.reshape(-1, 1), W2.T, b2.reshape(-1, 1))
    return oT.T


# speedup vs baseline: 5.4147x; 1.2095x over previous
import functools

import jax
import jax.numpy as jnp
from jax import lax
from jax.experimental import pallas as pl
from jax.experimental.pallas import tpu as pltpu
from jax.experimental.pallas import tpu_sc as plsc

EMBEDDING_DIM = 50
CONTEXT_SIZE = 5
NUM_CORES = 2
NUM_SUBCORES = 16
NUM_WORKERS = NUM_CORES * NUM_SUBCORES
CHUNK = 256
VEC = 16
FLAT_BLK = 524288


def _flatten_body(x_ref, o_ref):
    r = pl.program_id(2)
    o_ref[...] = x_ref[r]


@jax.jit
def _flatten(tabT):
    dim, v = tabT.shape
    vpad = 1 << (v - 1).bit_length()
    w = min(vpad, FLAT_BLK)
    halves = vpad // w
    d8s = (dim + 7) // 8
    return pl.pallas_call(
        _flatten_body,
        grid=(d8s, halves, 8),
        in_specs=[pl.BlockSpec((8, w), lambda d8, hh, r: (d8, hh))],
        out_specs=pl.BlockSpec((w,), lambda d8, hh, r: ((d8 * 8 + r) * halves + hh)),
        out_shape=jax.ShapeDtypeStruct((d8s * 8 * vpad,), jnp.float32),
    )(tabT)


@functools.partial(jax.jit, static_argnames=("n_chunks", "sizes"))
def _sc_gather_sum(emb_f, p_f, s_f, gi, pi, si, n_chunks, sizes):
    v_g, v_p, v_s = sizes
    dim = EMBEDDING_DIM
    tile = dim * CHUNK
    mesh = plsc.VectorSubcoreMesh(core_axis_name="c", subcore_axis_name="s")

    @functools.partial(
        pl.kernel,
        mesh=mesh,
        out_type=jax.ShapeDtypeStruct((NUM_WORKERS, n_chunks, tile),
                                      jnp.float32),
        compiler_params=pltpu.CompilerParams(use_tc_tiling_on_sc=False),
        scratch_types=[
            pltpu.VMEM((n_chunks, CHUNK), jnp.int32),
            pltpu.VMEM((n_chunks, CHUNK), jnp.int32),
            pltpu.VMEM((n_chunks, CHUNK), jnp.int32),
            pltpu.VMEM((tile,), jnp.int32),
            pltpu.VMEM((tile,), jnp.int32),
            pltpu.VMEM((tile,), jnp.int32),
            pltpu.VMEM((tile,), jnp.float32),
            pltpu.VMEM((tile,), jnp.float32),
            pltpu.VMEM((tile,), jnp.float32),
            pltpu.SemaphoreType.DMA,
            pltpu.SemaphoreType.DMA,
            pltpu.SemaphoreType.DMA,
            pltpu.SemaphoreType.DMA,
        ],
    )
    def k(emb_h, p_h, s_h, gi_h, pi_h, si_h, x_h,
          gi_v, pi_v, si_v, gix, pix, six, gbuf, pbuf, sbuf,
          sem0, sem1, sem2, sem3):
        wid = lax.axis_index("s") * NUM_CORES + lax.axis_index("c")
        pltpu.sync_copy(gi_h.at[wid], gi_v)
        pltpu.sync_copy(pi_h.at[wid], pi_v)
        pltpu.sync_copy(si_h.at[wid], si_v)

        def body(j, carry):
            def prep_body(d, _):
                for t in range(CHUNK // VEC):
                    src = pl.ds(t * VEC, VEC)
                    dst = pl.ds(d * CHUNK + t * VEC, VEC)
                    gix[dst] = gi_v[j, src] + d * v_g
                    pix[dst] = pi_v[j, src] + d * v_p
                    six[dst] = si_v[j, src] + d * v_s
                return _
            lax.fori_loop(0, dim, prep_body, 0)
            h0 = pltpu.async_copy(emb_h.at[gix], gbuf, sem0)
            h1 = pltpu.async_copy(p_h.at[pix], pbuf, sem1)
            h2 = pltpu.async_copy(s_h.at[six], sbuf, sem2)
            h0.wait()
            h1.wait()
            h2.wait()

            def sum_body(i, _):
                c = pl.ds(i * VEC, VEC)
                gbuf[c] = gbuf[c] + pbuf[c] + sbuf[c]
                return _
            lax.fori_loop(0, tile // VEC, sum_body, 0)
            o = pltpu.async_copy(gbuf, x_h.at[wid].at[j], sem3)
            o.wait()
            return carry

        lax.fori_loop(0, n_chunks, body, 0)

    return k(emb_f, p_f, s_f, gi, pi, si)


def _mlp_body(x_ref, w1_ref, b1_ref, w2_ref, b2_ref, o_ref):
    acc = None
    for c in range(CONTEXT_SIZE):
        part = jnp.dot(w1_ref[c], x_ref[c, 0],
                       preferred_element_type=jnp.float32)
        acc = part if acc is None else acc + part
    h = jnp.maximum(acc + b1_ref[...], 0.0)
    o = jnp.dot(w2_ref[...], h, preferred_element_type=jnp.float32)
    o = o + b2_ref[...]
    m = jnp.max(o, axis=0, keepdims=True)
    e = jnp.exp(o - m)
    lse = jnp.log(jnp.sum(e, axis=0, keepdims=True))
    o_ref[...] = (o - m) - lse


@jax.jit
def _tc_mlp(X5, W1t, b1, W2t, b2):
    ctx, n_tiles, dim, chunk = X5.shape
    hidden = W1t.shape[1]
    n_tags = W2t.shape[0]
    return pl.pallas_call(
        _mlp_body,
        grid=(n_tiles,),
        in_specs=[
            pl.BlockSpec((ctx, 1, dim, chunk), lambda i: (0, i, 0, 0)),
            pl.BlockSpec((ctx, hidden, dim), lambda i: (0, 0, 0)),
            pl.BlockSpec((hidden, 1), lambda i: (0, 0)),
            pl.BlockSpec((n_tags, hidden), lambda i: (0, 0)),
            pl.BlockSpec((n_tags, 1), lambda i: (0, 0)),
        ],
        out_specs=pl.BlockSpec((n_tags, chunk), lambda i: (0, i)),
        out_shape=jax.ShapeDtypeStruct((n_tags, n_tiles * chunk),
                                       jnp.float32),
    )(X5, W1t, b1, W2t, b2)


def kernel(inputs, p_inputs, s_inputs, emb, p_emb, s_emb, W1, b1, W2, b2):
    ctx, batch = inputs.shape
    dim = emb.shape[1]
    n_cols = ctx * batch
    per_w = n_cols // NUM_WORKERS
    n_chunks = per_w // CHUNK

    def prep(ix):
        return ix.reshape(NUM_WORKERS, n_chunks, CHUNK).astype(jnp.int32)

    def flat(tab):
        v = tab.shape[0]
        vpad = 1 << (v - 1).bit_length()
        return _flatten(tab.T), vpad

    emb_f, v_g = flat(emb)
    p_f, v_p = flat(p_emb)
    s_f, v_s = flat(s_emb)
    X = _sc_gather_sum(emb_f, p_f, s_f,
                       prep(inputs), prep(p_inputs), prep(s_inputs),
                       n_chunks, (v_g, v_p, v_s))
    X5 = X.reshape(ctx, batch // CHUNK, dim, CHUNK)
    W1t = W1.T.reshape(W1.shape[1], ctx, dim).transpose(1, 0, 2)
    oT = _tc_mlp(X5, W1t, b1.reshape(-1, 1), W2.T, b2.reshape(-1, 1))
    return oT.T
